# Initial kernel scaffold; baseline (speedup 1.0000x reference)
import jax
import jax.numpy as jnp
from jax.experimental import pallas as pl


def kernel(words, prefixes, suffixes, W_word, W_prefix, W_suffix):
    raise NotImplementedError("write your pallas kernel here")


# speedup vs baseline: 5.9050x; 5.9050x over previous
import functools

import jax
import jax.numpy as jnp
from jax import lax
from jax.experimental import pallas as pl
from jax.experimental.pallas import tpu as pltpu
from jax.experimental.pallas import tpu_sc as plsc

_B = 4096
_L = 50
_EMB = 128
_N = _B * _L
_NC = 2
_NS = 16
_NW = _NC * _NS
_R = _N // _NW
_C = 128
_NCH = _R // _C
_LANES = 16


def _emb_body(words_hbm, prefixes_hbm, suffixes_hbm,
              ww_hbm, wp_hbm, ws_hbm, out_hbm,
              widx, pidx, sidx, accw, bufp, bufs,
              sem0, sem1, sem2):
    wid = lax.axis_index("s") * _NC + lax.axis_index("c")
    base = wid * _R

    pltpu.sync_copy(words_hbm.at[pl.ds(base, _R)], widx)
    pltpu.sync_copy(prefixes_hbm.at[pl.ds(base, _R)], pidx)
    pltpu.sync_copy(suffixes_hbm.at[pl.ds(base, _R)], sidx)

    def chunk_body(g, carry):
        off = g * _C

        hw = pltpu.async_copy(ww_hbm.at[widx.at[pl.ds(off, _C)]], accw, sem0)
        hp = pltpu.async_copy(wp_hbm.at[pidx.at[pl.ds(off, _C)]], bufp, sem1)
        hs = pltpu.async_copy(ws_hbm.at[sidx.at[pl.ds(off, _C)]], bufs, sem2)
        hw.wait()
        hp.wait()
        hs.wait()

        def row_body(r, rc):
            for j in range(_EMB // _LANES):
                sl = pl.ds(j * _LANES, _LANES)
                accw[r, sl] = accw[r, sl] + bufp[r, sl] + bufs[r, sl]
            return rc

        lax.fori_loop(0, _C, row_body, 0, unroll=False)

        pltpu.sync_copy(accw, out_hbm.at[pl.ds(base + off, _C)])
        return carry

    lax.fori_loop(0, _NCH, chunk_body, 0, unroll=False)


@functools.partial(jax.jit, static_argnums=())
def _emb_call(words_f, prefixes_f, suffixes_f, ww, wp, ws):
    mesh = plsc.VectorSubcoreMesh(core_axis_name="c", subcore_axis_name="s")
    fn = pl.kernel(
        _emb_body,
        out_type=jax.ShapeDtypeStruct((_N, _EMB), jnp.float32),
        mesh=mesh,
        scratch_types=[
            pltpu.VMEM((_R,), jnp.int32),
            pltpu.VMEM((_R,), jnp.int32),
            pltpu.VMEM((_R,), jnp.int32),
            pltpu.VMEM((_C, _EMB), jnp.float32),
            pltpu.VMEM((_C, _EMB), jnp.float32),
            pltpu.VMEM((_C, _EMB), jnp.float32),
            pltpu.SemaphoreType.DMA,
            pltpu.SemaphoreType.DMA,
            pltpu.SemaphoreType.DMA,
        ],
    )
    return fn(words_f, prefixes_f, suffixes_f, ww, wp, ws)


def kernel(words, prefixes, suffixes, W_word, W_prefix, W_suffix):
    out = _emb_call(words.reshape(-1), prefixes.reshape(-1),
                    suffixes.reshape(-1), W_word, W_prefix, W_suffix)
    return out.reshape(_B, _L, _EMB)


# speedup vs baseline: 7.5950x; 1.2862x over previous
import functools

import jax
import jax.numpy as jnp
from jax import lax
from jax.experimental import pallas as pl
from jax.experimental.pallas import tpu as pltpu
from jax.experimental.pallas import tpu_sc as plsc

_B = 4096
_L = 50
_EMB = 128
_N = _B * _L
_NC = 2
_NS = 16
_NW = _NC * _NS
_R = _N // _NW
_C = 128
_NCH = _R // _C
_LANES = 16


def _emb_body(words_hbm, prefixes_hbm, suffixes_hbm,
              ww_hbm, wp_hbm, ws_hbm, out_hbm,
              widx, pidx, sidx,
              accw0, bufp0, bufs0, accw1, bufp1, bufs1,
              semw0, semp0, sems0, semw1, semp1, sems1):
    wid = lax.axis_index("s") * _NC + lax.axis_index("c")
    base = wid * _R

    accw = (accw0, accw1)
    bufp = (bufp0, bufp1)
    bufs = (bufs0, bufs1)
    semw = (semw0, semw1)
    semp = (semp0, semp1)
    sems = (sems0, sems1)

    pltpu.sync_copy(words_hbm.at[pl.ds(base, _R)], widx)
    pltpu.sync_copy(prefixes_hbm.at[pl.ds(base, _R)], pidx)
    pltpu.sync_copy(suffixes_hbm.at[pl.ds(base, _R)], sidx)

    def start_gathers(off, slot):
        pltpu.async_copy(ww_hbm.at[widx.at[pl.ds(off, _C)]], accw[slot],
                         semw[slot])
        pltpu.async_copy(wp_hbm.at[pidx.at[pl.ds(off, _C)]], bufp[slot],
                         semp[slot])
        pltpu.async_copy(ws_hbm.at[sidx.at[pl.ds(off, _C)]], bufs[slot],
                         sems[slot])

    def wait_gathers(off, slot):
        pltpu.make_async_copy(ww_hbm.at[widx.at[pl.ds(off, _C)]], accw[slot],
                              semw[slot]).wait()
        pltpu.make_async_copy(wp_hbm.at[pidx.at[pl.ds(off, _C)]], bufp[slot],
                              semp[slot]).wait()
        pltpu.make_async_copy(ws_hbm.at[sidx.at[pl.ds(off, _C)]], bufs[slot],
                              sems[slot]).wait()

    start_gathers(0, 0)

    def pair_body(g2, carry):
        for b in (0, 1):
            g = g2 * 2 + b
            off = g * _C
            wait_gathers(off, b)

            @pl.when(g < _NCH - 1)
            def _():
                start_gathers(off + _C, 1 - b)

            acc = accw[b]
            bp = bufp[b]
            bs = bufs[b]

            def row_body(r, rc, acc=acc, bp=bp, bs=bs):
                for j in range(_EMB // _LANES):
                    sl = pl.ds(j * _LANES, _LANES)
                    plsc.addupdate(acc.at[r, sl], bp[r, sl] + bs[r, sl])
                return rc

            lax.fori_loop(0, _C, row_body, 0, unroll=False)

            pltpu.sync_copy(acc, out_hbm.at[pl.ds(base + off, _C)])
        return carry

    lax.fori_loop(0, _NCH // 2, pair_body, 0, unroll=False)


@functools.partial(jax.jit, static_argnums=())
def _emb_call(words_f, prefixes_f, suffixes_f, ww, wp, ws):
    mesh = plsc.VectorSubcoreMesh(core_axis_name="c", subcore_axis_name="s")
    fn = pl.kernel(
        _emb_body,
        out_type=jax.ShapeDtypeStruct((_N, _EMB), jnp.float32),
        mesh=mesh,
        scratch_types=[
            pltpu.VMEM((_R,), jnp.int32),
            pltpu.VMEM((_R,), jnp.int32),
            pltpu.VMEM((_R,), jnp.int32),
            pltpu.VMEM((_C, _EMB), jnp.float32),
            pltpu.VMEM((_C, _EMB), jnp.float32),
            pltpu.VMEM((_C, _EMB), jnp.float32),
            pltpu.VMEM((_C, _EMB), jnp.float32),
            pltpu.VMEM((_C, _EMB), jnp.float32),
            pltpu.VMEM((_C, _EMB), jnp.float32),
            pltpu.SemaphoreType.DMA,
            pltpu.SemaphoreType.DMA,
            pltpu.SemaphoreType.DMA,
            pltpu.SemaphoreType.DMA,
            pltpu.SemaphoreType.DMA,
            pltpu.SemaphoreType.DMA,
        ],
    )
    return fn(words_f, prefixes_f, suffixes_f, ww, wp, ws)


def kernel(words, prefixes, suffixes, W_word, W_prefix, W_suffix):
    out = _emb_call(words.reshape(-1), prefixes.reshape(-1),
                    suffixes.reshape(-1), W_word, W_prefix, W_suffix)
    return out.reshape(_B, _L, _EMB)


# speedup vs baseline: 11.2338x; 1.4791x over previous
import functools

import jax
import jax.numpy as jnp
from jax import lax
from jax.experimental import pallas as pl
from jax.experimental.pallas import tpu as pltpu
from jax.experimental.pallas import tpu_sc as plsc

_B = 4096
_L = 50
_EMB = 128
_NC = 2
_NS = 16
_NW = _NC * _NS
_RB = _B // _NW
_CB = 2
_NCH = _RB // _CB
_LANES = 16


def _emb_body(words_hbm, prefixes_hbm, suffixes_hbm,
              ww_hbm, wp_hbm, ws_hbm, out_hbm,
              widx, pidx, sidx,
              accw0, bufp0, bufs0, accw1, bufp1, bufs1,
              semw0, semp0, sems0, semw1, semp1, sems1):
    wid = lax.axis_index("s") * _NC + lax.axis_index("c")
    b0 = wid * _RB

    accw = (accw0, accw1)
    bufp = (bufp0, bufp1)
    bufs = (bufs0, bufs1)
    semw = (semw0, semw1)
    semp = (semp0, semp1)
    sems = (sems0, sems1)

    pltpu.sync_copy(words_hbm.at[pl.ds(b0, _RB), :], widx)
    pltpu.sync_copy(prefixes_hbm.at[pl.ds(b0, _RB), :], pidx)
    pltpu.sync_copy(suffixes_hbm.at[pl.ds(b0, _RB), :], sidx)

    def start_gathers(boff, slot):
        for i in range(_CB):
            dsl = pl.ds(i * _L, _L)
            pltpu.async_copy(ww_hbm.at[widx.at[boff + i, :]],
                             accw[slot].at[dsl, :], semw[slot])
            pltpu.async_copy(wp_hbm.at[pidx.at[boff + i, :]],
                             bufp[slot].at[dsl, :], semp[slot])
            pltpu.async_copy(ws_hbm.at[sidx.at[boff + i, :]],
                             bufs[slot].at[dsl, :], sems[slot])

    def wait_gathers(boff, slot):
        for i in range(_CB):
            dsl = pl.ds(i * _L, _L)
            pltpu.make_async_copy(ww_hbm.at[widx.at[boff + i, :]],
                                  accw[slot].at[dsl, :], semw[slot]).wait()
            pltpu.make_async_copy(wp_hbm.at[pidx.at[boff + i, :]],
                                  bufp[slot].at[dsl, :], semp[slot]).wait()
            pltpu.make_async_copy(ws_hbm.at[sidx.at[boff + i, :]],
                                  bufs[slot].at[dsl, :], sems[slot]).wait()


    start_gathers(0, 0)

    def pair_body(g2, carry):
        for b in (0, 1):
            g = g2 * 2 + b
            boff = g * _CB
            wait_gathers(boff, b)

            @pl.when(g < _NCH - 1)
            def _():
                start_gathers(boff + _CB, 1 - b)

            acc = accw[b]
            bp = bufp[b]
            bs = bufs[b]

            def row_body(r, rc, acc=acc, bp=bp, bs=bs):
                for j in range(_EMB // _LANES):
                    sl = pl.ds(j * _LANES, _LANES)
                    plsc.addupdate(acc.at[r, sl], bp[r, sl] + bs[r, sl])
                return rc

            lax.fori_loop(0, _CB * _L, row_body, 0, unroll=False)

            for i in range(_CB):
                pltpu.sync_copy(acc.at[pl.ds(i * _L, _L), :],
                                out_hbm.at[b0 + boff + i])
        return carry

    lax.fori_loop(0, _NCH // 2, pair_body, 0, unroll=False)


@functools.partial(jax.jit, static_argnums=())
def _emb_call(words, prefixes, suffixes, ww, wp, ws):
    mesh = plsc.VectorSubcoreMesh(core_axis_name="c", subcore_axis_name="s")
    fn = pl.kernel(
        _emb_body,
        out_type=jax.ShapeDtypeStruct((_B, _L, _EMB), jnp.float32),
        mesh=mesh,
        scratch_types=[
            pltpu.VMEM((_RB, _L), jnp.int32),
            pltpu.VMEM((_RB, _L), jnp.int32),
            pltpu.VMEM((_RB, _L), jnp.int32),
            pltpu.VMEM((_CB * _L, _EMB), jnp.float32),
            pltpu.VMEM((_CB * _L, _EMB), jnp.float32),
            pltpu.VMEM((_CB * _L, _EMB), jnp.float32),
            pltpu.VMEM((_CB * _L, _EMB), jnp.float32),
            pltpu.VMEM((_CB * _L, _EMB), jnp.float32),
            pltpu.VMEM((_CB * _L, _EMB), jnp.float32),
            pltpu.SemaphoreType.DMA,
            pltpu.SemaphoreType.DMA,
            pltpu.SemaphoreType.DMA,
            pltpu.SemaphoreType.DMA,
            pltpu.SemaphoreType.DMA,
            pltpu.SemaphoreType.DMA,
        ],
    )
    return fn(words, prefixes, suffixes, ww, wp, ws)


def kernel(words, prefixes, suffixes, W_word, W_prefix, W_suffix):
    return _emb_call(words, prefixes, suffixes, W_word, W_prefix, W_suffix)


# speedup vs baseline: 11.2445x; 1.0010x over previous
import functools

import jax
import jax.numpy as jnp
from jax import lax
from jax.experimental import pallas as pl
from jax.experimental.pallas import tpu as pltpu
from jax.experimental.pallas import tpu_sc as plsc

_B = 4096
_L = 50
_EMB = 128
_NC = 2
_NS = 16
_NW = _NC * _NS
_RB = _B // _NW
_CB = 2
_NCH = _RB // _CB
_LANES = 16


def _emb_body(words_hbm, prefixes_hbm, suffixes_hbm,
              ww_hbm, wp_hbm, ws_hbm, out_hbm,
              widx, pidx, sidx,
              accw0, bufp0, bufs0, accw1, bufp1, bufs1,
              semw0, semp0, sems0, semw1, semp1, sems1):
    wid = lax.axis_index("s") * _NC + lax.axis_index("c")
    b0 = wid * _RB

    accw = (accw0, accw1)
    bufp = (bufp0, bufp1)
    bufs = (bufs0, bufs1)
    semw = (semw0, semw1)
    semp = (semp0, semp1)
    sems = (sems0, sems1)

    pltpu.sync_copy(words_hbm.at[pl.ds(b0, _RB), :], widx)
    pltpu.sync_copy(prefixes_hbm.at[pl.ds(b0, _RB), :], pidx)
    pltpu.sync_copy(suffixes_hbm.at[pl.ds(b0, _RB), :], sidx)

    def start_gathers(boff, slot):
        for i in range(_CB):
            dsl = pl.ds(i * _L, _L)
            pltpu.async_copy(ww_hbm.at[widx.at[boff + i, :]],
                             accw[slot].at[dsl, :], semw[slot])
            pltpu.async_copy(wp_hbm.at[pidx.at[boff + i, :]],
                             bufp[slot].at[dsl, :], semp[slot])
            pltpu.async_copy(ws_hbm.at[sidx.at[boff + i, :]],
                             bufs[slot].at[dsl, :], sems[slot])

    def wait_gathers(boff, slot):
        for i in range(_CB):
            dsl = pl.ds(i * _L, _L)
            pltpu.make_async_copy(ww_hbm.at[widx.at[boff + i, :]],
                                  accw[slot].at[dsl, :], semw[slot]).wait()
            pltpu.make_async_copy(wp_hbm.at[pidx.at[boff + i, :]],
                                  bufp[slot].at[dsl, :], semp[slot]).wait()
            pltpu.make_async_copy(ws_hbm.at[sidx.at[boff + i, :]],
                                  bufs[slot].at[dsl, :], sems[slot]).wait()


    start_gathers(0, 0)

    def pair_body(g2, carry):
        for b in (0, 1):
            g = g2 * 2 + b
            boff = g * _CB
            wait_gathers(boff, b)

            @pl.when(g < _NCH - 1)
            def _():
                start_gathers(boff + _CB, 1 - b)

            acc = accw[b]
            bp = bufp[b]
            bs = bufs[b]

            def row_body(r, rc, acc=acc, bp=bp, bs=bs):
                for j in range(_EMB // _LANES):
                    sl = pl.ds(j * _LANES, _LANES)
                    plsc.addupdate(acc.at[r, sl], bp[r, sl] + bs[r, sl])
                return rc

            lax.fori_loop(0, _CB * _L, row_body, 0, unroll=False)

            for i in range(_CB):
                pltpu.sync_copy(acc.at[pl.ds(i * _L, _L), :],
                                out_hbm.at[b0 + boff + i])
        return carry

    lax.fori_loop(0, _NCH // 2, pair_body, 0, unroll=False)


@functools.partial(jax.jit, static_argnums=())
def _emb_call(words, prefixes, suffixes, ww, wp, ws):
    mesh = plsc.VectorSubcoreMesh(core_axis_name="c", subcore_axis_name="s")
    fn = pl.kernel(
        _emb_body,
        out_type=jax.ShapeDtypeStruct((_B, _L, _EMB), jnp.float32),
        mesh=mesh,
        compiler_params=pltpu.CompilerParams(use_tc_tiling_on_sc=True),
        scratch_types=[
            pltpu.VMEM((_RB, _L), jnp.int32),
            pltpu.VMEM((_RB, _L), jnp.int32),
            pltpu.VMEM((_RB, _L), jnp.int32),
            pltpu.VMEM((_CB * _L, _EMB), jnp.float32),
            pltpu.VMEM((_CB * _L, _EMB), jnp.float32),
            pltpu.VMEM((_CB * _L, _EMB), jnp.float32),
            pltpu.VMEM((_CB * _L, _EMB), jnp.float32),
            pltpu.VMEM((_CB * _L, _EMB), jnp.float32),
            pltpu.VMEM((_CB * _L, _EMB), jnp.float32),
            pltpu.SemaphoreType.DMA,
            pltpu.SemaphoreType.DMA,
            pltpu.SemaphoreType.DMA,
            pltpu.SemaphoreType.DMA,
            pltpu.SemaphoreType.DMA,
            pltpu.SemaphoreType.DMA,
        ],
    )
    return fn(words, prefixes, suffixes, ww, wp, ws)


def kernel(words, prefixes, suffixes, W_word, W_prefix, W_suffix):
    return _emb_call(words, prefixes, suffixes, W_word, W_prefix, W_suffix)


# speedup vs baseline: 16.3845x; 1.4571x over previous
import functools

import jax
import jax.numpy as jnp
from jax import lax
from jax.experimental import pallas as pl
from jax.experimental.pallas import tpu as pltpu
from jax.experimental.pallas import tpu_sc as plsc

_B = 4096
_L = 50
_EMB = 128
_NC = 2
_NS = 16
_NW = _NC * _NS
_CB = _B // _NW
_LANES = 16


def _emb_body(words_hbm, prefixes_hbm, suffixes_hbm,
              ww_hbm, wp_hbm, ws_hbm, out_hbm,
              widx, pidx, sidx,
              accw0, bufp0, bufs0, accw1, bufp1, bufs1,
              semw0, semp0, sems0, semw1, semp1, sems1):
    wid = lax.axis_index("s") * _NC + lax.axis_index("c")
    b0 = wid * _CB

    accw = (accw0, accw1)
    bufp = (bufp0, bufp1)
    bufs = (bufs0, bufs1)
    semw = (semw0, semw1)
    semp = (semp0, semp1)
    sems = (sems0, sems1)

    pltpu.sync_copy(words_hbm.at[:, pl.ds(b0, _CB)], widx)
    pltpu.sync_copy(prefixes_hbm.at[:, pl.ds(b0, _CB)], pidx)
    pltpu.sync_copy(suffixes_hbm.at[:, pl.ds(b0, _CB)], sidx)

    def start_gathers(l, slot):
        pltpu.async_copy(ww_hbm.at[widx.at[l, :]], accw[slot], semw[slot])
        pltpu.async_copy(wp_hbm.at[pidx.at[l, :]], bufp[slot], semp[slot])
        pltpu.async_copy(ws_hbm.at[sidx.at[l, :]], bufs[slot], sems[slot])

    def wait_gathers(l, slot):
        pltpu.make_async_copy(ww_hbm.at[widx.at[l, :]], accw[slot],
                              semw[slot]).wait()
        pltpu.make_async_copy(wp_hbm.at[pidx.at[l, :]], bufp[slot],
                              semp[slot]).wait()
        pltpu.make_async_copy(ws_hbm.at[sidx.at[l, :]], bufs[slot],
                              sems[slot]).wait()

    start_gathers(0, 0)

    def pair_body(l2, carry):
        for b in (0, 1):
            l = l2 * 2 + b
            wait_gathers(l, b)

            @pl.when(l < _L - 1)
            def _():
                start_gathers(l + 1, 1 - b)

            acc = accw[b]
            bp = bufp[b]
            bs = bufs[b]

            def row_body(r, rc, acc=acc, bp=bp, bs=bs):
                for j in range(_EMB // _LANES):
                    sl = pl.ds(j * _LANES, _LANES)
                    plsc.addupdate(acc.at[r, sl], bp[r, sl] + bs[r, sl])
                return rc

            lax.fori_loop(0, _CB, row_body, 0, unroll=False)

            pltpu.sync_copy(acc, out_hbm.at[l, pl.ds(b0, _CB), :])
        return carry

    lax.fori_loop(0, _L // 2, pair_body, 0, unroll=False)


@functools.partial(jax.jit, static_argnums=())
def _emb_call(words_t, prefixes_t, suffixes_t, ww, wp, ws):
    mesh = plsc.VectorSubcoreMesh(core_axis_name="c", subcore_axis_name="s")
    fn = pl.kernel(
        _emb_body,
        out_type=jax.ShapeDtypeStruct((_L, _B, _EMB), jnp.float32),
        mesh=mesh,
        scratch_types=[
            pltpu.VMEM((_L, _CB), jnp.int32),
            pltpu.VMEM((_L, _CB), jnp.int32),
            pltpu.VMEM((_L, _CB), jnp.int32),
            pltpu.VMEM((_CB, _EMB), jnp.float32),
            pltpu.VMEM((_CB, _EMB), jnp.float32),
            pltpu.VMEM((_CB, _EMB), jnp.float32),
            pltpu.VMEM((_CB, _EMB), jnp.float32),
            pltpu.VMEM((_CB, _EMB), jnp.float32),
            pltpu.VMEM((_CB, _EMB), jnp.float32),
            pltpu.SemaphoreType.DMA,
            pltpu.SemaphoreType.DMA,
            pltpu.SemaphoreType.DMA,
            pltpu.SemaphoreType.DMA,
            pltpu.SemaphoreType.DMA,
            pltpu.SemaphoreType.DMA,
        ],
    )
    return fn(words_t, prefixes_t, suffixes_t, ww, wp, ws)


def kernel(words, prefixes, suffixes, W_word, W_prefix, W_suffix):
    out_t = _emb_call(words.T, prefixes.T, suffixes.T,
                      W_word, W_prefix, W_suffix)
    return out_t.transpose(1, 0, 2)
